# baseline (speedup 1.0000x reference)
import functools

import jax
import jax.numpy as jnp
from jax import lax
from jax.experimental import pallas as pl
from jax.experimental.pallas import tpu as pltpu
from jax.experimental.pallas import tpu_sc as plsc

D = 64
B = 4096
L = 200
NE = 1000000
NC, NS = 2, 16
NW = NC * NS
NROW = NE // 2
NBLK = NE // 128
CB = B // 128

K1_NBUF = 4
K1_TMAX = (NBLK + NW - 1) // NW
K1_ROUNDS = (K1_TMAX + K1_NBUF - 1) // K1_NBUF

K2_NBUF = 4
K2_ROUNDS = L // K2_NBUF

_mesh = plsc.VectorSubcoreMesh(core_axis_name="c", subcore_axis_name="s")


@functools.partial(
    pl.kernel,
    mesh=_mesh,
    out_type=jax.ShapeDtypeStruct((NROW, 128), jnp.float32),
    scratch_types=[
        pltpu.VMEM((K1_NBUF, 64, 128), jnp.float32),
        pltpu.VMEM((K1_NBUF, 64, 128), jnp.float32),
    ]
    + [pltpu.SemaphoreType.DMA] * (2 * K1_NBUF),
    compiler_params=pltpu.CompilerParams(
        use_tc_tiling_on_sc=True, needs_layout_passes=False
    ),
)
def _repack(tableT, tail2, tbl2, ibuf, obuf, *sems):
    gsem = sems[:K1_NBUF]
    ssem = sems[K1_NBUF:]
    wid = lax.axis_index("s") * NC + lax.axis_index("c")

    @pl.when(wid == 0)
    def _():
        pltpu.sync_copy(tail2, obuf.at[0, pl.ds(0, 32)])
        pltpu.sync_copy(obuf.at[0, pl.ds(0, 32)], tbl2.at[pl.ds(NE // 2 - 32, 32)])
    iota = lax.iota(jnp.int32, 16)
    riota = [iota + 16 * k for k in range(4)]

    def ebase(t):
        c = wid + NW * t
        return c, pl.multiple_of(c * 128, 128)

    def fire_in(t, b):
        c, be = ebase(t)

        @pl.when(c < NBLK)
        def _():
            pltpu.async_copy(tableT.at[:, pl.ds(be, 128)], ibuf.at[b], gsem[b])

    for b in range(K1_NBUF):
        fire_in(b, b)

    def round_body(g, carry):
        for b in range(K1_NBUF):
            t = g * K1_NBUF + b
            c, be = ebase(t)

            @pl.when(c < NBLK)
            def _process():
                pltpu.make_async_copy(
                    tableT.at[:, pl.ds(0, 128)], ibuf.at[b], gsem[b]
                ).wait()

                @pl.when(t >= K1_NBUF)
                def _():
                    pltpu.make_async_copy(
                        obuf.at[b], tbl2.at[pl.ds(0, 64)], ssem[b]
                    ).wait()

                @plsc.parallel_loop(0, 64, unroll=8)
                def jbody(J):
                    sj = 2 * J
                    vs = []
                    for p in range(2):
                        cidx = jnp.broadcast_to(sj + p, (16,)).astype(jnp.int32)
                        for k in range(4):
                            vs.append(
                                plsc.load_gather(ibuf.at[b], [riota[k], cidx])
                            )
                    for n, v in enumerate(vs):
                        p, k = divmod(n, 4)
                        obuf.at[b][J, pl.ds(p * 64 + 16 * k, 16)] = v
                rb = pl.multiple_of(be >> 1, 32)
                pltpu.async_copy(obuf.at[b], tbl2.at[pl.ds(rb, 64)], ssem[b])
                fire_in(t + K1_NBUF, b)

        return carry

    lax.fori_loop(0, K1_ROUNDS, round_body, 0)

    for b in range(K1_NBUF):
        t = K1_ROUNDS * K1_NBUF - K1_NBUF + b
        c, _ = ebase(t)

        @pl.when(c < NBLK)
        def _():
            pltpu.make_async_copy(
                obuf.at[b], tbl2.at[pl.ds(0, 64)], ssem[b]
            ).wait()


@functools.partial(
    pl.kernel,
    mesh=_mesh,
    out_type=jax.ShapeDtypeStruct((L, 8, CB, 8, 128), jnp.float32),
    scratch_types=[
        pltpu.VMEM((L, 128), jnp.int32),
        pltpu.VMEM((K2_NBUF, 128), jnp.int32),
        pltpu.VMEM((K2_NBUF, 128, 128), jnp.float32),
        pltpu.VMEM((K2_NBUF, 8, 8, 128), jnp.float32),
    ]
    + [pltpu.SemaphoreType.DMA] * (2 * K2_NBUF),
    compiler_params=pltpu.CompilerParams(needs_layout_passes=False),
)
def _gather(idxT3, tbl2, out5, idx_v, rows, gbuf, obuf, *sems):
    gsem = sems[:K2_NBUF]
    ssem = sems[K2_NBUF:]
    wid = lax.axis_index("s") * NC + lax.axis_index("c")
    iota = lax.iota(jnp.int32, 16)
    riota = [iota + 16 * g for g in range(8)]

    pltpu.sync_copy(idxT3.at[wid], idx_v)

    def prep_fire(l, b):
        for g in range(8):
            e = idx_v[l, pl.ds(16 * g, 16)]
            rows.at[b][pl.ds(16 * g, 16)] = e >> 1
        pltpu.async_copy(tbl2.at[rows.at[b]], gbuf.at[b], gsem[b])

    for b in range(K2_NBUF):
        prep_fire(b, b)

    def round_body(g, carry):
        for b in range(K2_NBUF):
            l = g * K2_NBUF + b
            pltpu.make_async_copy(
                tbl2.at[rows.at[b]], gbuf.at[b], gsem[b]
            ).wait()

            @pl.when(l >= K2_NBUF)
            def _():
                pltpu.make_async_copy(
                    obuf.at[b], out5.at[0, :, 0], ssem[b]
                ).wait()

            pcols = [
                ((idx_v[l, pl.ds(16 * gg, 16)] & 1) << 6) for gg in range(8)
            ]

            @plsc.parallel_loop(0, 64, unroll=8)
            def dbody(d):
                r = d >> 3
                i = d & 7
                vs = [
                    plsc.load_gather(gbuf.at[b], [riota[gg], pcols[gg] + d])
                    for gg in range(8)
                ]
                for gg, v in enumerate(vs):
                    obuf.at[b][r, i, pl.ds(16 * gg, 16)] = v
            pltpu.async_copy(obuf.at[b], out5.at[l, :, wid], ssem[b])

            @pl.when(l + K2_NBUF < L)
            def _():
                prep_fire(l + K2_NBUF, b)

        return carry

    lax.fori_loop(0, K2_ROUNDS, round_body, 0)

    for b in range(K2_NBUF):
        pltpu.make_async_copy(obuf.at[b], out5.at[0, :, 0], ssem[b]).wait()


def kernel(entity_pairs, table):
    idxT3 = entity_pairs[:, :, 0].reshape(CB, 128, L).transpose(0, 2, 1)
    tail2 = table[NE - 64 :].reshape(32, 128)
    tbl2 = _repack(table.T, tail2)
    out5 = _gather(idxT3, tbl2)
    return out5.transpose(2, 4, 0, 1, 3).reshape(B, L, D)


# speedup vs baseline: 1.3096x; 1.3096x over previous
import functools

import jax
import jax.numpy as jnp
from jax import lax
from jax.experimental import pallas as pl
from jax.experimental.pallas import tpu as pltpu
from jax.experimental.pallas import tpu_sc as plsc

D = 64
B = 4096
L = 200
N = B * L
NC, NS = 2, 16
NW = NC * NS
PER_W = N // NW
CHUNK = 128
NCHUNK = PER_W // CHUNK
NBUF = 8
PF = 4
ROUNDS = NCHUNK // NBUF


@functools.partial(
    pl.kernel,
    mesh=plsc.VectorSubcoreMesh(core_axis_name="c", subcore_axis_name="s"),
    out_type=jax.ShapeDtypeStruct((N, D), jnp.float32),
    scratch_types=[
        pltpu.VMEM((NCHUNK, CHUNK), jnp.int32),
        pltpu.VMEM((NBUF, CHUNK, D), jnp.float32),
    ]
    + [pltpu.SemaphoreType.DMA] * (2 * NBUF),
    compiler_params=pltpu.CompilerParams(use_tc_tiling_on_sc=False),
)
def _gather_kernel(idx_hbm, table_hbm, out_hbm, idx_v, bufs, *sems):
    gsem = sems[:NBUF]
    ssem = sems[NBUF:]
    wid = lax.axis_index("s") * NC + lax.axis_index("c")
    base = wid * PER_W

    pltpu.sync_copy(idx_hbm.at[wid], idx_v)

    for b in range(PF):
        pltpu.async_copy(table_hbm.at[idx_v.at[b]], bufs.at[b], gsem[b])

    def round_body(g, carry):
        for b in range(NBUF):
            c = g * NBUF + b
            bp = (b + PF) % NBUF

            @pl.when(c + PF < NCHUNK)
            def _():
                @pl.when(c >= NBUF - PF)
                def _():
                    pltpu.make_async_copy(
                        bufs.at[bp], out_hbm.at[pl.ds(base, CHUNK)], ssem[bp]
                    ).wait()

                pltpu.async_copy(
                    table_hbm.at[idx_v.at[c + PF]], bufs.at[bp], gsem[bp]
                )

            pltpu.make_async_copy(
                table_hbm.at[idx_v.at[0]], bufs.at[b], gsem[b]
            ).wait()
            pltpu.async_copy(
                bufs.at[b], out_hbm.at[pl.ds(base + c * CHUNK, CHUNK)], ssem[b]
            )

        return carry

    lax.fori_loop(0, ROUNDS, round_body, 0)

    for b in range(NBUF):
        pltpu.make_async_copy(
            bufs.at[b], out_hbm.at[pl.ds(base, CHUNK)], ssem[b]
        ).wait()


def kernel(entity_pairs, table):
    idx = entity_pairs[:, :, 0].reshape(NW, NCHUNK, CHUNK)
    out = _gather_kernel(idx, table)
    return out.reshape(B, L, D)
